# baseline (speedup 1.0000x reference)
import functools

import jax
import jax.numpy as jnp
from jax import lax
from jax.experimental import pallas as pl
from jax.experimental.pallas import tpu as pltpu
from jax.experimental.pallas import tpu_sc as plsc

BATCH = 16384
EMBED_DIM = 32
NUM_CORES = 2
NUM_SUBCORES = 16
NUM_WORKERS = NUM_CORES * NUM_SUBCORES
B_PER_W = BATCH // NUM_WORKERS
CHUNK = 128
NCHUNK = B_PER_W // CHUNK
LANES = 16


def _mf_kernel(users, items, user_embed, item_embed, out,
               idx_u, idx_i, u_rows, v_rows, out_v, sem):
    wid = lax.axis_index("s") * NUM_CORES + lax.axis_index("c")
    base = wid * B_PER_W

    for j in range(NCHUNK):
        pltpu.sync_copy(users.at[pl.ds(base + j * CHUNK, CHUNK)], idx_u.at[j])
        pltpu.sync_copy(items.at[pl.ds(base + j * CHUNK, CHUNK)], idx_i.at[j])

    copies = []
    for j in range(NCHUNK):
        copies.append(pltpu.async_copy(
            user_embed.at[idx_u.at[j]], u_rows.at[pl.ds(j * CHUNK, CHUNK)], sem))
        copies.append(pltpu.async_copy(
            item_embed.at[idx_i.at[j]], v_rows.at[pl.ds(j * CHUNK, CHUNK)], sem))
    for c in copies:
        c.wait()

    lane = lax.iota(jnp.int32, 16)

    def body(g, _):
        acc = jnp.zeros((16,), jnp.float32)
        for j in range(LANES):
            r = g * LANES + j
            w = (u_rows[r, pl.ds(0, 16)] * v_rows[r, pl.ds(0, 16)]
                 + u_rows[r, pl.ds(16, 16)] * v_rows[r, pl.ds(16, 16)])
            acc = jnp.where(lane == j, jnp.sum(w), acc)
        out_v[pl.ds(pl.multiple_of(g * LANES, LANES), LANES)] = acc
        return 0

    lax.fori_loop(0, B_PER_W // LANES, body, 0)

    pltpu.sync_copy(out_v, out.at[pl.ds(base, B_PER_W)])


@jax.jit
def kernel(users, items, user_embed, item_embed):
    mesh = plsc.VectorSubcoreMesh(core_axis_name="c", subcore_axis_name="s")
    f = pl.kernel(
        _mf_kernel,
        out_type=jax.ShapeDtypeStruct((BATCH,), jnp.float32),
        mesh=mesh,
        compiler_params=pltpu.CompilerParams(
            needs_layout_passes=False, use_tc_tiling_on_sc=False),
        scratch_types=[
            pltpu.VMEM((NCHUNK, CHUNK), jnp.int32),
            pltpu.VMEM((NCHUNK, CHUNK), jnp.int32),
            pltpu.VMEM((B_PER_W, EMBED_DIM), jnp.float32),
            pltpu.VMEM((B_PER_W, EMBED_DIM), jnp.float32),
            pltpu.VMEM((B_PER_W,), jnp.float32),
            pltpu.SemaphoreType.DMA,
        ],
    )
    return f(users.astype(jnp.int32), items.astype(jnp.int32),
             user_embed, item_embed)


# speedup vs baseline: 4.3053x; 4.3053x over previous
import jax
import jax.numpy as jnp
from jax import lax
from jax.experimental import pallas as pl
from jax.experimental.pallas import tpu as pltpu
from jax.experimental.pallas import tpu_sc as plsc

BATCH = 16384
EMBED_DIM = 32
NUM_CORES = 2
NUM_SUBCORES = 16
NUM_WORKERS = NUM_CORES * NUM_SUBCORES
B_PER_W = BATCH // NUM_WORKERS
LANES = 16
BLK = 128
DEPTH = 8


def _mf_kernel(users, items, ut, vt, out,
               idx_u, idx_i, dim_iota, bufs_u, bufs_v, out_v, sem_u, sem_v):
    wid = lax.axis_index("s") * NUM_CORES + lax.axis_index("c")
    base = wid * B_PER_W

    iota16 = lax.iota(jnp.int32, 16)
    dim_iota[pl.ds(0, 16)] = iota16
    dim_iota[pl.ds(16, 16)] = iota16 + 16

    pltpu.sync_copy(users.at[pl.ds(base, B_PER_W)], idx_u.at[pl.ds(0, B_PER_W)])
    pltpu.sync_copy(items.at[pl.ds(base, B_PER_W)], idx_i.at[pl.ds(0, B_PER_W)])

    def fire(k, slot):
        ju = idx_u[pl.ds(k, 16)][0]
        cu = pl.multiple_of((ju >> 7) << 7, BLK)
        pltpu.async_copy(ut.at[dim_iota, pl.ds(cu, BLK)],
                         bufs_u.at[slot], sem_u)
        ji = idx_i[pl.ds(k, 16)][0]
        ci = pl.multiple_of((ji >> 7) << 7, BLK)
        pltpu.async_copy(vt.at[dim_iota, pl.ds(ci, BLK)],
                         bufs_v.at[slot], sem_v)

    def drain(slot):
        pltpu.make_async_copy(ut.at[dim_iota, pl.ds(0, BLK)],
                              bufs_u.at[slot], sem_u).wait()
        pltpu.make_async_copy(vt.at[dim_iota, pl.ds(0, BLK)],
                              bufs_v.at[slot], sem_v).wait()

    for kk in range(DEPTH):
        fire(kk, kk)

    n_groups = B_PER_W // DEPTH

    def body(g, acc):
        for kk in range(DEPTH):
            k = g * DEPTH + kk
            drain(kk)
            ju = idx_u[pl.ds(k, 16)][0]
            ji = idx_i[pl.ds(k, 16)][0]
            cu = jnp.broadcast_to(ju & 127, (16,))
            ci = jnp.broadcast_to(ji & 127, (16,))
            u0 = plsc.load_gather(bufs_u.at[kk], [iota16, cu])
            u1 = plsc.load_gather(bufs_u.at[kk], [iota16 + 16, cu])
            v0 = plsc.load_gather(bufs_v.at[kk], [iota16, ci])
            v1 = plsc.load_gather(bufs_v.at[kk], [iota16 + 16, ci])
            w = u0 * v0 + u1 * v1
            acc = jnp.where(iota16 == (k & 15), jnp.sum(w), acc)

            @pl.when(g < n_groups - 1)
            def _():
                fire(k + DEPTH, kk)

            if kk == DEPTH - 1:
                @pl.when((g & 1) == 1)
                def _():
                    out_v[pl.ds(pl.multiple_of((g >> 1) * 16, 16), 16)] = acc
        return acc

    lax.fori_loop(0, n_groups, body, jnp.zeros((16,), jnp.float32))

    pltpu.sync_copy(out_v, out.at[pl.ds(base, B_PER_W)])


@jax.jit
def kernel(users, items, user_embed, item_embed):
    mesh = plsc.VectorSubcoreMesh(core_axis_name="c", subcore_axis_name="s")
    f = pl.kernel(
        _mf_kernel,
        out_type=jax.ShapeDtypeStruct((BATCH,), jnp.float32),
        mesh=mesh,
        compiler_params=pltpu.CompilerParams(needs_layout_passes=False),
        scratch_types=[
            pltpu.VMEM((B_PER_W + 16,), jnp.int32),
            pltpu.VMEM((B_PER_W + 16,), jnp.int32),
            pltpu.VMEM((EMBED_DIM,), jnp.int32),
            pltpu.VMEM((DEPTH, EMBED_DIM, BLK), jnp.float32),
            pltpu.VMEM((DEPTH, EMBED_DIM, BLK), jnp.float32),
            pltpu.VMEM((B_PER_W,), jnp.float32),
            pltpu.SemaphoreType.DMA,
            pltpu.SemaphoreType.DMA,
        ],
    )
    return f(users.astype(jnp.int32), items.astype(jnp.int32),
             user_embed.T, item_embed.T)


# speedup vs baseline: 4.4638x; 1.0368x over previous
import jax
import jax.numpy as jnp
from jax import lax
from jax.experimental import pallas as pl
from jax.experimental.pallas import tpu as pltpu
from jax.experimental.pallas import tpu_sc as plsc

BATCH = 16384
EMBED_DIM = 32
NUM_CORES = 2
NUM_SUBCORES = 16
NUM_WORKERS = NUM_CORES * NUM_SUBCORES
B_PER_W = BATCH // NUM_WORKERS
LANES = 16
BLK = 128
DEPTH = 8


def _mf_kernel(users, items, ut, vt, out,
               idx_u, idx_i, dim_iota, bufs_u, bufs_v, out_v, sem_u, sem_v):
    wid = lax.axis_index("s") * NUM_CORES + lax.axis_index("c")
    base = wid * B_PER_W

    iota16 = lax.iota(jnp.int32, 16)
    dim_iota[pl.ds(0, 16)] = iota16
    dim_iota[pl.ds(16, 16)] = iota16 + 16

    pltpu.sync_copy(users.at[pl.ds(base, B_PER_W)], idx_u.at[pl.ds(0, B_PER_W)])
    pltpu.sync_copy(items.at[pl.ds(base, B_PER_W)], idx_i.at[pl.ds(0, B_PER_W)])

    def fire(k, slot):
        ju = idx_u[pl.ds(k, 16)][0]
        cu = pl.multiple_of((ju >> 7) << 7, BLK)
        pltpu.async_copy(ut.at[:, pl.ds(cu, BLK)],
                         bufs_u.at[slot], sem_u)
        ji = idx_i[pl.ds(k, 16)][0]
        ci = pl.multiple_of((ji >> 7) << 7, BLK)
        pltpu.async_copy(vt.at[:, pl.ds(ci, BLK)],
                         bufs_v.at[slot], sem_v)

    def drain(slot):
        pltpu.make_async_copy(ut.at[:, pl.ds(0, BLK)],
                              bufs_u.at[slot], sem_u).wait()
        pltpu.make_async_copy(vt.at[:, pl.ds(0, BLK)],
                              bufs_v.at[slot], sem_v).wait()

    for kk in range(DEPTH):
        fire(kk, kk)

    n_groups = B_PER_W // DEPTH

    def body(g, acc):
        for kk in range(DEPTH):
            k = g * DEPTH + kk
            drain(kk)
            ju = idx_u[pl.ds(k, 16)][0]
            ji = idx_i[pl.ds(k, 16)][0]
            cu = jnp.broadcast_to(ju & 127, (16,))
            ci = jnp.broadcast_to(ji & 127, (16,))
            u0 = plsc.load_gather(bufs_u.at[kk], [iota16, cu])
            u1 = plsc.load_gather(bufs_u.at[kk], [iota16 + 16, cu])
            v0 = plsc.load_gather(bufs_v.at[kk], [iota16, ci])
            v1 = plsc.load_gather(bufs_v.at[kk], [iota16 + 16, ci])
            w = u0 * v0 + u1 * v1
            acc = jnp.where(iota16 == (k & 15), jnp.sum(w), acc)

            @pl.when(g < n_groups - 1)
            def _():
                fire(k + DEPTH, kk)

            if kk == DEPTH - 1:
                @pl.when((g & 1) == 1)
                def _():
                    out_v[pl.ds(pl.multiple_of((g >> 1) * 16, 16), 16)] = acc
        return acc

    lax.fori_loop(0, n_groups, body, jnp.zeros((16,), jnp.float32))

    pltpu.sync_copy(out_v, out.at[pl.ds(base, B_PER_W)])


@jax.jit
def kernel(users, items, user_embed, item_embed):
    mesh = plsc.VectorSubcoreMesh(core_axis_name="c", subcore_axis_name="s")
    f = pl.kernel(
        _mf_kernel,
        out_type=jax.ShapeDtypeStruct((BATCH,), jnp.float32),
        mesh=mesh,
        compiler_params=pltpu.CompilerParams(needs_layout_passes=False),
        scratch_types=[
            pltpu.VMEM((B_PER_W + 16,), jnp.int32),
            pltpu.VMEM((B_PER_W + 16,), jnp.int32),
            pltpu.VMEM((EMBED_DIM,), jnp.int32),
            pltpu.VMEM((DEPTH, EMBED_DIM, BLK), jnp.float32),
            pltpu.VMEM((DEPTH, EMBED_DIM, BLK), jnp.float32),
            pltpu.VMEM((B_PER_W,), jnp.float32),
            pltpu.SemaphoreType.DMA,
            pltpu.SemaphoreType.DMA,
        ],
    )
    return f(users.astype(jnp.int32), items.astype(jnp.int32),
             user_embed.T, item_embed.T)


# speedup vs baseline: 4.4845x; 1.0046x over previous
import jax
import jax.numpy as jnp
from jax import lax
from jax.experimental import pallas as pl
from jax.experimental.pallas import tpu as pltpu
from jax.experimental.pallas import tpu_sc as plsc

BATCH = 16384
EMBED_DIM = 32
NUM_CORES = 2
NUM_SUBCORES = 16
NUM_WORKERS = NUM_CORES * NUM_SUBCORES
B_PER_W = BATCH // NUM_WORKERS
LANES = 16
BLK = 128
DEPTH = 8


def _mf_kernel(users, items, ut, vt, out,
               idx_u, idx_i, dim_iota, bufs_u, bufs_v, out_v,
               sem_u, sem_v, sem_u2, sem_v2):
    wid = lax.axis_index("s") * NUM_CORES + lax.axis_index("c")
    base = wid * B_PER_W

    iota16 = lax.iota(jnp.int32, 16)
    dim_iota[pl.ds(0, 16)] = iota16
    dim_iota[pl.ds(16, 16)] = iota16 + 16

    pltpu.sync_copy(users.at[pl.ds(base, B_PER_W)], idx_u.at[pl.ds(0, B_PER_W)])
    pltpu.sync_copy(items.at[pl.ds(base, B_PER_W)], idx_i.at[pl.ds(0, B_PER_W)])

    HALF = EMBED_DIM // 2

    def fire(k, slot):
        ju = idx_u[pl.ds(k, 16)][0]
        cu = pl.multiple_of((ju >> 7) << 7, BLK)
        pltpu.async_copy(ut.at[pl.ds(0, HALF), pl.ds(cu, BLK)],
                         bufs_u.at[slot, pl.ds(0, HALF)], sem_u)
        pltpu.async_copy(ut.at[pl.ds(HALF, HALF), pl.ds(cu, BLK)],
                         bufs_u.at[slot, pl.ds(HALF, HALF)], sem_u2)
        ji = idx_i[pl.ds(k, 16)][0]
        ci = pl.multiple_of((ji >> 7) << 7, BLK)
        pltpu.async_copy(vt.at[pl.ds(0, HALF), pl.ds(ci, BLK)],
                         bufs_v.at[slot, pl.ds(0, HALF)], sem_v)
        pltpu.async_copy(vt.at[pl.ds(HALF, HALF), pl.ds(ci, BLK)],
                         bufs_v.at[slot, pl.ds(HALF, HALF)], sem_v2)

    def drain(slot):
        pltpu.make_async_copy(ut.at[pl.ds(0, HALF), pl.ds(0, BLK)],
                              bufs_u.at[slot, pl.ds(0, HALF)], sem_u).wait()
        pltpu.make_async_copy(ut.at[pl.ds(HALF, HALF), pl.ds(0, BLK)],
                              bufs_u.at[slot, pl.ds(HALF, HALF)], sem_u2).wait()
        pltpu.make_async_copy(vt.at[pl.ds(0, HALF), pl.ds(0, BLK)],
                              bufs_v.at[slot, pl.ds(0, HALF)], sem_v).wait()
        pltpu.make_async_copy(vt.at[pl.ds(HALF, HALF), pl.ds(0, BLK)],
                              bufs_v.at[slot, pl.ds(HALF, HALF)], sem_v2).wait()

    for kk in range(DEPTH):
        fire(kk, kk)

    n_groups = B_PER_W // DEPTH

    def body(g, acc):
        for kk in range(DEPTH):
            k = g * DEPTH + kk
            drain(kk)
            ju = idx_u[pl.ds(k, 16)][0]
            ji = idx_i[pl.ds(k, 16)][0]
            cu = jnp.broadcast_to(ju & 127, (16,))
            ci = jnp.broadcast_to(ji & 127, (16,))
            u0 = plsc.load_gather(bufs_u.at[kk], [iota16, cu])
            u1 = plsc.load_gather(bufs_u.at[kk], [iota16 + 16, cu])
            v0 = plsc.load_gather(bufs_v.at[kk], [iota16, ci])
            v1 = plsc.load_gather(bufs_v.at[kk], [iota16 + 16, ci])
            w = u0 * v0 + u1 * v1
            acc = jnp.where(iota16 == (k & 15), jnp.sum(w), acc)

            @pl.when(g < n_groups - 1)
            def _():
                fire(k + DEPTH, kk)

            if kk == DEPTH - 1:
                @pl.when((g & 1) == 1)
                def _():
                    out_v[pl.ds(pl.multiple_of((g >> 1) * 16, 16), 16)] = acc
        return acc

    lax.fori_loop(0, n_groups, body, jnp.zeros((16,), jnp.float32))

    pltpu.sync_copy(out_v, out.at[pl.ds(base, B_PER_W)])


@jax.jit
def kernel(users, items, user_embed, item_embed):
    mesh = plsc.VectorSubcoreMesh(core_axis_name="c", subcore_axis_name="s")
    f = pl.kernel(
        _mf_kernel,
        out_type=jax.ShapeDtypeStruct((BATCH,), jnp.float32),
        mesh=mesh,
        compiler_params=pltpu.CompilerParams(needs_layout_passes=False),
        scratch_types=[
            pltpu.VMEM((B_PER_W + 16,), jnp.int32),
            pltpu.VMEM((B_PER_W + 16,), jnp.int32),
            pltpu.VMEM((EMBED_DIM,), jnp.int32),
            pltpu.VMEM((DEPTH, EMBED_DIM, BLK), jnp.float32),
            pltpu.VMEM((DEPTH, EMBED_DIM, BLK), jnp.float32),
            pltpu.VMEM((B_PER_W,), jnp.float32),
            pltpu.SemaphoreType.DMA,
            pltpu.SemaphoreType.DMA,
            pltpu.SemaphoreType.DMA,
            pltpu.SemaphoreType.DMA,
        ],
    )
    return f(users.astype(jnp.int32), items.astype(jnp.int32),
             user_embed.T, item_embed.T)
